# Initial kernel scaffold; baseline (speedup 1.0000x reference)
import jax
import jax.numpy as jnp
from jax.experimental import pallas as pl


def kernel(positions, level_embedding, position_in_level_embedding, sinusoidal_table):
    raise NotImplementedError("write your pallas kernel here")


# speedup vs baseline: 5.5197x; 5.5197x over previous
import functools

import jax
import jax.numpy as jnp
from jax import lax
from jax.experimental import pallas as pl
from jax.experimental.pallas import tpu as pltpu
from jax.experimental.pallas import tpu_sc as plsc

EMBED = 128
TABLE_ROWS = 32
N_TOTAL = 3276800


def _fuse_kernel(w_ref, out_ref):
    r = lax.broadcasted_iota(jnp.int32, (TABLE_ROWS, 64), 0)
    c = lax.broadcasted_iota(jnp.int32, (TABLE_ROWS, 64), 1)
    sel = (c == r // 8) | (c == 8 + r % 8) | (c == 24 + r)
    onehot = sel.astype(jnp.float32)
    out_ref[...] = jnp.dot(onehot, w_ref[...], preferred_element_type=jnp.float32)


def _build_fused_table(level_emb, pos_emb, sin_table):
    w = jnp.zeros((64, EMBED), jnp.float32)
    w = w.at[0:4, 0:32].set(level_emb)
    w = w.at[8:17, 32:64].set(pos_emb)
    w = w.at[24:49, 64:128].set(sin_table)
    return pl.pallas_call(
        _fuse_kernel,
        out_shape=jax.ShapeDtypeStruct((TABLE_ROWS, EMBED), jnp.float32),
    )(w)


def _gather_body(n_chunks, chunk, fused_hbm, pos_hbm, out_hbm, idx_v,
                 rows_v, sem):
    info = plsc.get_sparse_core_info()
    nc = info.num_cores
    wid = lax.axis_index("s") * nc + lax.axis_index("c")
    per_w = n_chunks * chunk
    base = wid * per_w

    def step(i, _):
        off = base + i * chunk
        pltpu.sync_copy(pos_hbm.at[pl.ds(off, chunk)], idx_v)
        pltpu.async_copy(fused_hbm.at[idx_v], rows_v, sem).wait()
        pltpu.sync_copy(rows_v, out_hbm.at[pl.ds(off, chunk)])
        return 0

    lax.fori_loop(0, n_chunks, step, 0)


def _sc_gather(fused, positions):
    n = positions.shape[0]
    info = plsc.get_sparse_core_info()
    nw = info.num_cores * info.num_subcores
    chunk = 128
    n_chunks = n // (nw * chunk)
    assert n_chunks * nw * chunk == n
    mesh = plsc.VectorSubcoreMesh(core_axis_name="c", subcore_axis_name="s")
    grid_kernel = pl.kernel(
        functools.partial(_gather_body, n_chunks, chunk),
        out_type=jax.ShapeDtypeStruct((n, EMBED), jnp.float32),
        mesh=mesh,
        scratch_types=[
            pltpu.VMEM((chunk,), jnp.int32),
            pltpu.VMEM((chunk, EMBED), jnp.float32),
            pltpu.SemaphoreType.DMA,
        ],
    )
    return grid_kernel(fused, positions)


def kernel(positions, level_embedding, position_in_level_embedding, sinusoidal_table):
    positions = positions.astype(jnp.int32)
    fused = _build_fused_table(level_embedding, position_in_level_embedding,
                               sinusoidal_table)
    return _sc_gather(fused, positions)


# speedup vs baseline: 61.2078x; 11.0890x over previous
import functools

import jax
import jax.numpy as jnp
from jax import lax
from jax.experimental import pallas as pl
from jax.experimental.pallas import tpu as pltpu
from jax.experimental.pallas import tpu_sc as plsc

EMBED = 128
TABLE_ROWS = 32
N_TOTAL = 3276800


def _fuse_kernel(w_ref, out_ref):
    r = lax.broadcasted_iota(jnp.int32, (TABLE_ROWS, 64), 0)
    c = lax.broadcasted_iota(jnp.int32, (TABLE_ROWS, 64), 1)
    sel = (c == r // 8) | (c == 8 + r % 8) | (c == 24 + r)
    onehot = sel.astype(jnp.float32)
    out_ref[...] = jnp.dot(onehot, w_ref[...], preferred_element_type=jnp.float32)


def _build_fused_table(level_emb, pos_emb, sin_table):
    w = jnp.zeros((64, EMBED), jnp.float32)
    w = w.at[0:4, 0:32].set(level_emb)
    w = w.at[8:17, 32:64].set(pos_emb)
    w = w.at[24:49, 64:128].set(sin_table)
    return pl.pallas_call(
        _fuse_kernel,
        out_shape=jax.ShapeDtypeStruct((TABLE_ROWS, EMBED), jnp.float32),
    )(w)


def _gather_body(n_chunks, chunk, fused_hbm, pos_hbm, out_hbm,
                 idx_a, idx_b, rows_a, rows_b, table_sh,
                 sem_ia, sem_ib, sem_g, sem_oa, sem_ob):
    info = plsc.get_sparse_core_info()
    nc = info.num_cores
    sid = lax.axis_index("s")
    wid = sid * nc + lax.axis_index("c")
    per_w = n_chunks * chunk
    base = wid * per_w

    @pl.when(sid == 0)
    def _():
        pltpu.sync_copy(fused_hbm, table_sh)

    plsc.subcore_barrier()

    def start_idx(i, idx_v, sem):
        off = base + jnp.minimum(i, n_chunks - 1) * chunk
        pltpu.make_async_copy(pos_hbm.at[pl.ds(off, chunk)], idx_v, sem).start()

    def wait_idx(idx_v, sem):
        pltpu.make_async_copy(pos_hbm.at[pl.ds(base, chunk)], idx_v, sem).wait()

    def wait_out(rows_v, sem):
        pltpu.make_async_copy(rows_v, out_hbm.at[pl.ds(base, chunk)], sem).wait()

    def handle(i, idx_v, rows_v, sem_i, sem_o, idx_nv, sem_in):
        wait_idx(idx_v, sem_i)

        @pl.when(i >= 2)
        def _():
            wait_out(rows_v, sem_o)

        gather = pltpu.async_copy(table_sh.at[idx_v], rows_v, sem_g)
        start_idx(i + 1, idx_nv, sem_in)
        gather.wait()
        pltpu.make_async_copy(
            rows_v, out_hbm.at[pl.ds(base + i * chunk, chunk)], sem_o
        ).start()

    start_idx(0, idx_a, sem_ia)

    def superstep(j, _):
        handle(2 * j, idx_a, rows_a, sem_ia, sem_oa, idx_b, sem_ib)
        handle(2 * j + 1, idx_b, rows_b, sem_ib, sem_ob, idx_a, sem_ia)
        return 0

    lax.fori_loop(0, n_chunks // 2, superstep, 0)

    wait_out(rows_a, sem_oa)
    wait_out(rows_b, sem_ob)
    wait_idx(idx_a, sem_ia)


def _sc_gather(fused, positions):
    n = positions.shape[0]
    info = plsc.get_sparse_core_info()
    nw = info.num_cores * info.num_subcores
    chunk = 128
    n_chunks = n // (nw * chunk)
    assert n_chunks * nw * chunk == n and n_chunks % 2 == 0
    mesh = plsc.VectorSubcoreMesh(core_axis_name="c", subcore_axis_name="s")
    grid_kernel = pl.kernel(
        functools.partial(_gather_body, n_chunks, chunk),
        out_type=jax.ShapeDtypeStruct((n, EMBED), jnp.float32),
        mesh=mesh,
        scratch_types=[
            pltpu.VMEM((chunk,), jnp.int32),
            pltpu.VMEM((chunk,), jnp.int32),
            pltpu.VMEM((chunk, EMBED), jnp.float32),
            pltpu.VMEM((chunk, EMBED), jnp.float32),
            pltpu.VMEM_SHARED((TABLE_ROWS, EMBED), jnp.float32),
            pltpu.SemaphoreType.DMA,
            pltpu.SemaphoreType.DMA,
            pltpu.SemaphoreType.DMA,
            pltpu.SemaphoreType.DMA,
            pltpu.SemaphoreType.DMA,
        ],
    )
    return grid_kernel(fused, positions)


def kernel(positions, level_embedding, position_in_level_embedding, sinusoidal_table):
    positions = positions.astype(jnp.int32)
    fused = _build_fused_table(level_embedding, position_in_level_embedding,
                               sinusoidal_table)
    return _sc_gather(fused, positions)
